# Initial kernel scaffold; baseline (speedup 1.0000x reference)
import jax
import jax.numpy as jnp
from jax.experimental import pallas as pl


def kernel(feature, edge_index, edge_type, W1_l, W1_r, b1, W2_l, W2_r, b2):
    raise NotImplementedError("write your pallas kernel here")


# speedup vs baseline: 17.2352x; 17.2352x over previous
import functools

import jax
import jax.numpy as jnp
from jax import lax
from jax.experimental import pallas as pl
from jax.experimental.pallas import tpu as pltpu
from jax.experimental.pallas import tpu_sc as plsc

N = 100000
E = 1600000
D_IN = 16
D_HID = 64
Z_W = 16
HR_W = 8
NP = 100352

NC = 2
NS = 16
NW = NC * NS
CHUNK = 125
NCH = E // CHUNK
RW = E // (NW * CHUNK)
K = 8
T = RW // K

PER_TILE = NP // NS

ROWBLK = 2048
NBLK = NP // ROWBLK


def _seg_body(width, with_deg, x_hbm, edges_hbm, *refs):
    if with_deg:
        (zagg_hbm, ones_hbm, zdeg_hbm,
         agg_out, deg_out, src_buf, dst_buf, rows, ones,
         agg_sh, deg_sh, gsem, ssem) = refs
    else:
        (zagg_hbm,
         agg_out, src_buf, dst_buf, rows,
         agg_sh, gsem, ssem) = refs
        deg_out = deg_sh = ones = None

    c = lax.axis_index("c")
    s = lax.axis_index("s")
    w = c * NS + s
    base = s * PER_TILE

    pltpu.sync_copy(zagg_hbm, agg_sh.at[pl.ds(base, PER_TILE)])
    if with_deg:
        pltpu.sync_copy(ones_hbm, ones)
        pltpu.sync_copy(zdeg_hbm, deg_sh.at[pl.ds(base, PER_TILE)])

    plsc.subcore_barrier()

    row0 = w * RW

    def outer(t, _):
        r = row0 + t * K
        pltpu.sync_copy(edges_hbm.at[0, pl.ds(r, K)], src_buf)
        pltpu.sync_copy(edges_hbm.at[1, pl.ds(r, K)], dst_buf)
        gd = [pltpu.async_copy(x_hbm.at[src_buf.at[j]], rows.at[j], gsem)
              for j in range(K)]
        for d in gd:
            d.wait()
        sd = []
        for j in range(K):
            sd.append(pltpu.async_copy(
                rows.at[j], agg_sh.at[dst_buf.at[j]], ssem, add=True))
            if with_deg:
                sd.append(pltpu.async_copy(
                    ones, deg_sh.at[dst_buf.at[j]], ssem, add=True))
        for d in sd:
            d.wait()
        return 0

    lax.fori_loop(0, T, outer, 0)

    plsc.subcore_barrier()

    pltpu.sync_copy(agg_sh.at[pl.ds(base, PER_TILE)],
                    agg_out.at[c, pl.ds(base, PER_TILE)])
    if with_deg:
        pltpu.sync_copy(deg_sh.at[pl.ds(base, PER_TILE)],
                        deg_out.at[pl.ds(c * NP + base, PER_TILE)])


def _make_seg_kernel(width, with_deg):
    mesh = plsc.VectorSubcoreMesh(core_axis_name="c", subcore_axis_name="s")
    out_type = [jax.ShapeDtypeStruct((NC, NP, width), jnp.float32)]
    if with_deg:
        out_type.append(jax.ShapeDtypeStruct((NC * NP,), jnp.float32))
    scratch = [
        pltpu.VMEM((K, CHUNK), jnp.int32),
        pltpu.VMEM((K, CHUNK), jnp.int32),
        pltpu.VMEM((K, CHUNK, width), jnp.float32),
    ]
    if with_deg:
        scratch += [
            pltpu.VMEM((CHUNK,), jnp.float32),
        ]
    scratch += [pltpu.VMEM_SHARED((NP, width), jnp.float32)]
    if with_deg:
        scratch += [pltpu.VMEM_SHARED((NP,), jnp.float32)]
    scratch += [pltpu.SemaphoreType.DMA, pltpu.SemaphoreType.DMA]

    return pl.kernel(
        functools.partial(_seg_body, width, with_deg),
        out_type=tuple(out_type),
        mesh=mesh,
        scratch_types=scratch,
        compiler_params=pltpu.CompilerParams(use_tc_tiling_on_sc=False),
    )


def _dense1_body(pref, dref, xref, w1l, w1r, b1r, w2l, w2r, b2r, zref, hrref):
    deg = jnp.maximum(dref[0] + dref[1], 1.0)
    inv = 1.0 / deg
    mean = (pref[0] + pref[1]) * inv
    h = jnp.dot(mean, w1l[...], preferred_element_type=jnp.float32)
    h += jnp.dot(xref[...], w1r[...], preferred_element_type=jnp.float32)
    h = jnp.maximum(h + b1r[...], 0.0)
    zref[...] = jnp.dot(h, w2l[...], preferred_element_type=jnp.float32)
    hrref[...] = jnp.dot(h, w2r[...], preferred_element_type=jnp.float32) + b2r[...]


def _dense2_body(aref, dref, hrref, oref):
    deg = jnp.maximum(dref[0] + dref[1], 1.0)
    inv = 1.0 / deg
    a = (aref[0] + aref[1])[:, :HR_W]
    oref[...] = a * inv + hrref[...]


_full = lambda *dims: pl.BlockSpec(dims, lambda i: (0,) * len(dims))


def kernel(feature, edge_index, edge_type, W1_l, W1_r, b1, W2_l, W2_r, b2):
    del edge_type
    edges2d = edge_index.reshape(2, NCH, CHUNK)

    zagg16 = jnp.zeros((PER_TILE, D_IN), jnp.float32)
    ones_e = jnp.ones((CHUNK,), jnp.float32)
    zeros_d = jnp.zeros((PER_TILE,), jnp.float32)
    agg1_p, deg_flat = _make_seg_kernel(D_IN, True)(
        feature, edges2d, zagg16, ones_e, zeros_d)
    deg_p = deg_flat.reshape(NC, NP, 1)

    w2l_p = jnp.zeros((D_HID, Z_W), jnp.float32).at[:, :3].set(W2_l)
    w2r_p = jnp.zeros((D_HID, HR_W), jnp.float32).at[:, :3].set(W2_r)
    b1r = b1.reshape(1, D_HID)
    b2r = jnp.zeros((1, HR_W), jnp.float32).at[0, :3].set(b2)

    z, hr = pl.pallas_call(
        _dense1_body,
        grid=(NBLK,),
        in_specs=[
            pl.BlockSpec((NC, ROWBLK, D_IN), lambda i: (0, i, 0)),
            pl.BlockSpec((NC, ROWBLK, 1), lambda i: (0, i, 0)),
            pl.BlockSpec((ROWBLK, D_IN), lambda i: (i, 0)),
            _full(D_IN, D_HID), _full(D_IN, D_HID), _full(1, D_HID),
            _full(D_HID, Z_W), _full(D_HID, HR_W), _full(1, HR_W),
        ],
        out_specs=[
            pl.BlockSpec((ROWBLK, Z_W), lambda i: (i, 0)),
            pl.BlockSpec((ROWBLK, HR_W), lambda i: (i, 0)),
        ],
        out_shape=[
            jax.ShapeDtypeStruct((N, Z_W), jnp.float32),
            jax.ShapeDtypeStruct((N, HR_W), jnp.float32),
        ],
    )(agg1_p, deg_p, feature, W1_l, W1_r, b1r, w2l_p, w2r_p, b2r)

    zagg_z = jnp.zeros((PER_TILE, Z_W), jnp.float32)
    (agg2_p,) = _make_seg_kernel(Z_W, False)(z, edges2d, zagg_z)

    out8 = pl.pallas_call(
        _dense2_body,
        grid=(NBLK,),
        in_specs=[
            pl.BlockSpec((NC, ROWBLK, Z_W), lambda i: (0, i, 0)),
            pl.BlockSpec((NC, ROWBLK, 1), lambda i: (0, i, 0)),
            pl.BlockSpec((ROWBLK, HR_W), lambda i: (i, 0)),
        ],
        out_specs=pl.BlockSpec((ROWBLK, HR_W), lambda i: (i, 0)),
        out_shape=jax.ShapeDtypeStruct((N, HR_W), jnp.float32),
    )(agg2_p, deg_p, hr)

    return out8[:, :3]


# speedup vs baseline: 24.6600x; 1.4308x over previous
import functools

import jax
import jax.numpy as jnp
from jax import lax
from jax.experimental import pallas as pl
from jax.experimental.pallas import tpu as pltpu
from jax.experimental.pallas import tpu_sc as plsc

N = 100000
E = 1600000
D_IN = 16
D_HID = 64
Z_W = 16
HR_W = 8
NP = 100352

NC = 2
NS = 16
NW = NC * NS
CHUNK = 125
NCH = E // CHUNK
RW = E // (NW * CHUNK)
K = 8
T = RW // K

PER_TILE = NP // NS

ROWBLK = 2048
NBLK = NP // ROWBLK


def _seg_body(width, with_deg, x_hbm, edges_hbm, *refs):
    if with_deg:
        (zagg_hbm, ones_hbm, zdeg_hbm,
         agg_out, deg_out, src_buf, dst_buf, rows, ones,
         agg_sh, deg_sh, gsem, ssem) = refs
    else:
        (zagg_hbm,
         agg_out, src_buf, dst_buf, rows,
         agg_sh, gsem, ssem) = refs
        deg_out = deg_sh = ones = None

    c = lax.axis_index("c")
    s = lax.axis_index("s")
    w = c * NS + s
    base = s * PER_TILE

    pltpu.sync_copy(zagg_hbm, agg_sh.at[pl.ds(base, PER_TILE)])
    if with_deg:
        pltpu.sync_copy(ones_hbm, ones)
        pltpu.sync_copy(zdeg_hbm, deg_sh.at[pl.ds(base, PER_TILE)])

    plsc.subcore_barrier()

    row0 = w * RW

    def outer(t, _):
        r = row0 + t * K
        pltpu.sync_copy(edges_hbm.at[0, pl.ds(r, K)], src_buf)
        pltpu.sync_copy(edges_hbm.at[1, pl.ds(r, K)], dst_buf)
        gd = [pltpu.async_copy(x_hbm.at[src_buf.at[j]], rows.at[j], gsem)
              for j in range(K)]
        for d in gd:
            d.wait()
        sd = []
        for j in range(K):
            sd.append(pltpu.async_copy(
                rows.at[j], agg_sh.at[dst_buf.at[j]], ssem, add=True))
            if with_deg:
                sd.append(pltpu.async_copy(
                    ones, deg_sh.at[dst_buf.at[j]], ssem, add=True))
        for d in sd:
            d.wait()
        return 0

    lax.fori_loop(0, T, outer, 0)

    plsc.subcore_barrier()

    pltpu.sync_copy(agg_sh.at[pl.ds(base, PER_TILE)],
                    agg_out.at[c, pl.ds(base, PER_TILE)])
    if with_deg:
        pltpu.sync_copy(deg_sh.at[pl.ds(base, PER_TILE)],
                        deg_out.at[pl.ds(c * NP + base, PER_TILE)])


def _make_seg_kernel(width, with_deg):
    mesh = plsc.VectorSubcoreMesh(core_axis_name="c", subcore_axis_name="s")
    out_type = [jax.ShapeDtypeStruct((NC, NP, width), jnp.float32)]
    if with_deg:
        out_type.append(jax.ShapeDtypeStruct((NC * NP,), jnp.float32))
    scratch = [
        pltpu.VMEM((K, CHUNK), jnp.int32),
        pltpu.VMEM((K, CHUNK), jnp.int32),
        pltpu.VMEM((K, CHUNK, width), jnp.float32),
    ]
    if with_deg:
        scratch += [
            pltpu.VMEM((CHUNK,), jnp.float32),
        ]
    scratch += [pltpu.VMEM_SHARED((NP, width), jnp.float32)]
    if with_deg:
        scratch += [pltpu.VMEM_SHARED((NP,), jnp.float32)]
    scratch += [pltpu.SemaphoreType.DMA, pltpu.SemaphoreType.DMA]

    return pl.kernel(
        functools.partial(_seg_body, width, with_deg),
        out_type=tuple(out_type),
        mesh=mesh,
        scratch_types=scratch,
        compiler_params=pltpu.CompilerParams(use_tc_tiling_on_sc=False),
    )


def _dense1_body(pref, iref, xref, w1l, w1r, b1r, w2l, w2r, b2r, zref, hrref):
    mean = (pref[0] + pref[1]) * iref[...]
    h = jnp.dot(mean, w1l[...], preferred_element_type=jnp.float32)
    h += jnp.dot(xref[...], w1r[...], preferred_element_type=jnp.float32)
    h = jnp.maximum(h + b1r[...], 0.0)
    zref[...] = jnp.dot(h, w2l[...], preferred_element_type=jnp.float32)
    hrref[...] = jnp.dot(h, w2r[...], preferred_element_type=jnp.float32) + b2r[...]


def _dense2_body(aref, iref, hrref, sel, oref):
    mean = (aref[0] + aref[1]) * iref[...]
    oref[...] = jnp.dot(mean, sel[...],
                        preferred_element_type=jnp.float32) + hrref[...]


_full = lambda *dims: pl.BlockSpec(dims, lambda i: (0,) * len(dims))


def kernel(feature, edge_index, edge_type, W1_l, W1_r, b1, W2_l, W2_r, b2):
    del edge_type
    edges2d = edge_index.reshape(2, NCH, CHUNK)

    zagg16 = jnp.zeros((PER_TILE, D_IN), jnp.float32)
    ones_e = jnp.ones((CHUNK,), jnp.float32)
    zeros_d = jnp.zeros((PER_TILE,), jnp.float32)
    agg1_p, deg_flat = _make_seg_kernel(D_IN, True)(
        feature, edges2d, zagg16, ones_e, zeros_d)
    a1r = agg1_p.reshape(NC, NP * D_IN // 128, 128)
    deg = deg_flat.reshape(NC, NP).sum(axis=0)
    invr = jnp.repeat(1.0 / jnp.maximum(deg, 1.0), D_IN
                      ).reshape(NP * D_IN // 128, 128)
    xr = feature.reshape(N * D_IN // 128, 128)

    w2l_p = jnp.zeros((D_HID, Z_W), jnp.float32).at[:, :3].set(W2_l)
    w2r_p = jnp.zeros((D_HID, HR_W), jnp.float32).at[:, :3].set(W2_r)
    b1r = b1.reshape(1, D_HID)
    b2r = jnp.zeros((1, HR_W), jnp.float32).at[0, :3].set(b2)

    PKB = ROWBLK * D_IN // 128
    eye8 = jnp.eye(8, dtype=jnp.float32)
    w1l_b = jnp.kron(eye8, W1_l)
    w1r_b = jnp.kron(eye8, W1_r)
    b1_b = jnp.tile(b1, 8).reshape(1, 8 * D_HID)
    w2l_b = jnp.kron(eye8, w2l_p)
    w2r_b = jnp.kron(eye8, w2r_p)
    b2_b = jnp.tile(b2r[0], 8).reshape(1, 8 * HR_W)
    z_pk, hr_pk = pl.pallas_call(
        _dense1_body,
        grid=(NBLK,),
        in_specs=[
            pl.BlockSpec((NC, PKB, 128), lambda i: (0, i, 0)),
            pl.BlockSpec((PKB, 128), lambda i: (i, 0)),
            pl.BlockSpec((PKB, 128), lambda i: (i, 0)),
            _full(128, 8 * D_HID), _full(128, 8 * D_HID), _full(1, 8 * D_HID),
            _full(8 * D_HID, 128), _full(8 * D_HID, 8 * HR_W),
            _full(1, 8 * HR_W),
        ],
        out_specs=[
            pl.BlockSpec((PKB, 128), lambda i: (i, 0)),
            pl.BlockSpec((PKB, 8 * HR_W), lambda i: (i, 0)),
        ],
        out_shape=[
            jax.ShapeDtypeStruct((NP * D_IN // 128, 128), jnp.float32),
            jax.ShapeDtypeStruct((NP * D_IN // 128, 8 * HR_W), jnp.float32),
        ],
    )(a1r, invr, xr, w1l_b, w1r_b, b1_b, w2l_b, w2r_b, b2_b)
    z = z_pk.reshape(NP, Z_W)

    zagg_z = jnp.zeros((PER_TILE, Z_W), jnp.float32)
    (agg2_p,) = _make_seg_kernel(Z_W, False)(z, edges2d, zagg_z)

    a2r = agg2_p.reshape(NC, NP * Z_W // 128, 128)
    sel = jnp.kron(eye8, jnp.eye(Z_W, HR_W, dtype=jnp.float32))
    out_pk = pl.pallas_call(
        _dense2_body,
        grid=(NBLK,),
        in_specs=[
            pl.BlockSpec((NC, PKB, 128), lambda i: (0, i, 0)),
            pl.BlockSpec((PKB, 128), lambda i: (i, 0)),
            pl.BlockSpec((PKB, 8 * HR_W), lambda i: (i, 0)),
            _full(128, 8 * HR_W),
        ],
        out_specs=pl.BlockSpec((PKB, 8 * HR_W), lambda i: (i, 0)),
        out_shape=jax.ShapeDtypeStruct((NP * D_IN // 128, 8 * HR_W),
                                       jnp.float32),
    )(a2r, invr, hr_pk, sel)

    return out_pk.reshape(NP, HR_W)[:N, :3]


# speedup vs baseline: 24.7374x; 1.0031x over previous
import functools

import jax
import jax.numpy as jnp
from jax import lax
from jax.experimental import pallas as pl
from jax.experimental.pallas import tpu as pltpu
from jax.experimental.pallas import tpu_sc as plsc

N = 100000
E = 1600000
D_IN = 16
D_HID = 64
Z_W = 8
HR_W = 8
NP = 100352

NC = 2
NS = 16
NW = NC * NS
CHUNK = 125
NCH = E // CHUNK
RW = E // (NW * CHUNK)
K = 8
T = RW // K

PER_TILE = NP // NS

ROWBLK = 2048
NBLK = NP // ROWBLK


def _seg_body(width, with_deg, x_hbm, edges_hbm, *refs):
    if with_deg:
        (zagg_hbm, ones_hbm, zdeg_hbm,
         agg_out, deg_out, src_buf, dst_buf, rows, ones,
         agg_sh, deg_sh, gsem, ssem) = refs
    else:
        (zagg_hbm,
         agg_out, src_buf, dst_buf, rows,
         agg_sh, gsem, ssem) = refs
        deg_out = deg_sh = ones = None

    c = lax.axis_index("c")
    s = lax.axis_index("s")
    w = c * NS + s
    base = s * PER_TILE

    pltpu.sync_copy(zagg_hbm, agg_sh.at[pl.ds(base, PER_TILE)])
    if with_deg:
        pltpu.sync_copy(ones_hbm, ones)
        pltpu.sync_copy(zdeg_hbm, deg_sh.at[pl.ds(base, PER_TILE)])

    plsc.subcore_barrier()

    row0 = w * RW

    def outer(t, _):
        r = row0 + t * K
        pltpu.sync_copy(edges_hbm.at[0, pl.ds(r, K)], src_buf)
        pltpu.sync_copy(edges_hbm.at[1, pl.ds(r, K)], dst_buf)
        gd = [pltpu.async_copy(x_hbm.at[src_buf.at[j]], rows.at[j], gsem)
              for j in range(K)]
        for d in gd:
            d.wait()
        sd = []
        for j in range(K):
            sd.append(pltpu.async_copy(
                rows.at[j], agg_sh.at[dst_buf.at[j]], ssem, add=True))
            if with_deg:
                sd.append(pltpu.async_copy(
                    ones, deg_sh.at[dst_buf.at[j]], ssem, add=True))
        for d in sd:
            d.wait()
        return 0

    lax.fori_loop(0, T, outer, 0)

    plsc.subcore_barrier()

    pltpu.sync_copy(agg_sh.at[pl.ds(base, PER_TILE)],
                    agg_out.at[c, pl.ds(base, PER_TILE)])
    if with_deg:
        pltpu.sync_copy(deg_sh.at[pl.ds(base, PER_TILE)],
                        deg_out.at[pl.ds(c * NP + base, PER_TILE)])


def _make_seg_kernel(width, with_deg):
    mesh = plsc.VectorSubcoreMesh(core_axis_name="c", subcore_axis_name="s")
    out_type = [jax.ShapeDtypeStruct((NC, NP, width), jnp.float32)]
    if with_deg:
        out_type.append(jax.ShapeDtypeStruct((NC * NP,), jnp.float32))
    scratch = [
        pltpu.VMEM((K, CHUNK), jnp.int32),
        pltpu.VMEM((K, CHUNK), jnp.int32),
        pltpu.VMEM((K, CHUNK, width), jnp.float32),
    ]
    if with_deg:
        scratch += [
            pltpu.VMEM((CHUNK,), jnp.float32),
        ]
    scratch += [pltpu.VMEM_SHARED((NP, width), jnp.float32)]
    if with_deg:
        scratch += [pltpu.VMEM_SHARED((NP,), jnp.float32)]
    scratch += [pltpu.SemaphoreType.DMA, pltpu.SemaphoreType.DMA]

    return pl.kernel(
        functools.partial(_seg_body, width, with_deg),
        out_type=tuple(out_type),
        mesh=mesh,
        scratch_types=scratch,
        compiler_params=pltpu.CompilerParams(use_tc_tiling_on_sc=False),
    )


def _dense1_body(pref, iref, xref, w1l, w1r, b1r, w2l, w2r, b2r, zref, hrref):
    mean = (pref[0] + pref[1]) * iref[...]
    h = jnp.dot(mean, w1l[...], preferred_element_type=jnp.float32)
    h += jnp.dot(xref[...], w1r[...], preferred_element_type=jnp.float32)
    h = jnp.maximum(h + b1r[...], 0.0)
    zref[...] = jnp.dot(h, w2l[...], preferred_element_type=jnp.float32)
    hrref[...] = jnp.dot(h, w2r[...], preferred_element_type=jnp.float32) + b2r[...]


def _dense2_body(aref, iref, hrref, oref):
    mean = (aref[0] + aref[1]) * iref[...]
    oref[...] = mean + hrref[...]


_full = lambda *dims: pl.BlockSpec(dims, lambda i: (0,) * len(dims))


def kernel(feature, edge_index, edge_type, W1_l, W1_r, b1, W2_l, W2_r, b2):
    del edge_type
    edges2d = edge_index.reshape(2, NCH, CHUNK)

    zagg16 = jnp.zeros((PER_TILE, D_IN), jnp.float32)
    ones_e = jnp.ones((CHUNK,), jnp.float32)
    zeros_d = jnp.zeros((PER_TILE,), jnp.float32)
    agg1_p, deg_flat = _make_seg_kernel(D_IN, True)(
        feature, edges2d, zagg16, ones_e, zeros_d)
    a1r = agg1_p.reshape(NC, NP * D_IN // 128, 128)
    deg = deg_flat.reshape(NC, NP).sum(axis=0)
    invr = jnp.repeat(1.0 / jnp.maximum(deg, 1.0), D_IN
                      ).reshape(NP * D_IN // 128, 128)
    xr = feature.reshape(N * D_IN // 128, 128)

    w2l_p = jnp.zeros((D_HID, Z_W), jnp.float32).at[:, :3].set(W2_l)
    w2r_p = jnp.zeros((D_HID, HR_W), jnp.float32).at[:, :3].set(W2_r)
    b1r = b1.reshape(1, D_HID)
    b2r = jnp.zeros((1, HR_W), jnp.float32).at[0, :3].set(b2)

    PKB = ROWBLK * D_IN // 128
    eye8 = jnp.eye(8, dtype=jnp.float32)
    w1l_b = jnp.kron(eye8, W1_l)
    w1r_b = jnp.kron(eye8, W1_r)
    b1_b = jnp.tile(b1, 8).reshape(1, 8 * D_HID)
    w2l_b = jnp.kron(eye8, w2l_p)
    w2r_b = jnp.kron(eye8, w2r_p)
    b2_b = jnp.tile(b2r[0], 8).reshape(1, 8 * HR_W)
    z_pk, hr_pk = pl.pallas_call(
        _dense1_body,
        grid=(NBLK,),
        in_specs=[
            pl.BlockSpec((NC, PKB, 128), lambda i: (0, i, 0)),
            pl.BlockSpec((PKB, 128), lambda i: (i, 0)),
            pl.BlockSpec((PKB, 128), lambda i: (i, 0)),
            _full(128, 8 * D_HID), _full(128, 8 * D_HID), _full(1, 8 * D_HID),
            _full(8 * D_HID, 8 * Z_W), _full(8 * D_HID, 8 * HR_W),
            _full(1, 8 * HR_W),
        ],
        out_specs=[
            pl.BlockSpec((PKB, 8 * Z_W), lambda i: (i, 0)),
            pl.BlockSpec((PKB, 8 * HR_W), lambda i: (i, 0)),
        ],
        out_shape=[
            jax.ShapeDtypeStruct((NP * D_IN // 128, 8 * Z_W), jnp.float32),
            jax.ShapeDtypeStruct((NP * D_IN // 128, 8 * HR_W), jnp.float32),
        ],
    )(a1r, invr, xr, w1l_b, w1r_b, b1_b, w2l_b, w2r_b, b2_b)
    z = z_pk.reshape(NP, Z_W)

    zagg_z = jnp.zeros((PER_TILE, Z_W), jnp.float32)
    (agg2_p,) = _make_seg_kernel(Z_W, False)(z, edges2d, zagg_z)

    a2r = agg2_p.reshape(NC, NP * D_IN // 128, 8 * Z_W)
    invr8 = jnp.repeat(1.0 / jnp.maximum(deg, 1.0), Z_W
                       ).reshape(NP * D_IN // 128, 8 * Z_W)
    out_pk = pl.pallas_call(
        _dense2_body,
        grid=(NBLK,),
        in_specs=[
            pl.BlockSpec((NC, PKB, 8 * Z_W), lambda i: (0, i, 0)),
            pl.BlockSpec((PKB, 8 * Z_W), lambda i: (i, 0)),
            pl.BlockSpec((PKB, 8 * HR_W), lambda i: (i, 0)),
        ],
        out_specs=pl.BlockSpec((PKB, 8 * HR_W), lambda i: (i, 0)),
        out_shape=jax.ShapeDtypeStruct((NP * D_IN // 128, 8 * HR_W),
                                       jnp.float32),
    )(a2r, invr8, hr_pk)

    return out_pk.reshape(NP, HR_W)[:N, :3]


# speedup vs baseline: 26.2685x; 1.0619x over previous
import functools

import jax
import jax.numpy as jnp
from jax import lax
from jax.experimental import pallas as pl
from jax.experimental.pallas import tpu as pltpu
from jax.experimental.pallas import tpu_sc as plsc

N = 100000
E = 1600000
D_IN = 16
D_HID = 64
Z_W = 8
HR_W = 8
NP = 100352

NC = 2
NS = 16
NW = NC * NS
CHUNK = 125
NCH = E // CHUNK
RW = E // (NW * CHUNK)
K1 = 8
K2 = 16

PER_TILE = NP // NS

ROWBLK = 2048
NBLK = NP // ROWBLK


def _seg_body(width, with_deg, K, x_hbm, edges_hbm, *refs):
    if with_deg:
        (zagg_hbm, ones_hbm, zdeg_hbm,
         agg_out, deg_out, src_buf, dst_buf, rows, ones,
         agg_sh, deg_sh, gsem, ssem) = refs
    else:
        (zagg_hbm,
         agg_out, src_buf, dst_buf, rows,
         agg_sh, gsem, ssem) = refs
        deg_out = deg_sh = ones = None

    c = lax.axis_index("c")
    s = lax.axis_index("s")
    w = c * NS + s
    base = s * PER_TILE

    pltpu.sync_copy(zagg_hbm, agg_sh.at[pl.ds(base, PER_TILE)])
    if with_deg:
        pltpu.sync_copy(ones_hbm, ones)
        pltpu.sync_copy(zdeg_hbm, deg_sh.at[pl.ds(base, PER_TILE)])

    plsc.subcore_barrier()

    row0 = w * RW
    T = RW // K

    def outer(t, _):
        r = row0 + t * K
        pltpu.sync_copy(edges_hbm.at[0, pl.ds(r, K)], src_buf)
        pltpu.sync_copy(edges_hbm.at[1, pl.ds(r, K)], dst_buf)
        gd = [pltpu.async_copy(x_hbm.at[src_buf.at[j]], rows.at[j], gsem)
              for j in range(K)]
        for d in gd:
            d.wait()
        sd = []
        for j in range(K):
            sd.append(pltpu.async_copy(
                rows.at[j], agg_sh.at[dst_buf.at[j]], ssem, add=True))
            if with_deg:
                sd.append(pltpu.async_copy(
                    ones, deg_sh.at[dst_buf.at[j]], ssem, add=True))
        for d in sd:
            d.wait()
        return 0

    lax.fori_loop(0, T, outer, 0)

    plsc.subcore_barrier()

    pltpu.sync_copy(agg_sh.at[pl.ds(base, PER_TILE)],
                    agg_out.at[c, pl.ds(base, PER_TILE)])
    if with_deg:
        pltpu.sync_copy(deg_sh.at[pl.ds(base, PER_TILE)],
                        deg_out.at[pl.ds(c * NP + base, PER_TILE)])


def _make_seg_kernel(width, with_deg, K):
    mesh = plsc.VectorSubcoreMesh(core_axis_name="c", subcore_axis_name="s")
    out_type = [jax.ShapeDtypeStruct((NC, NP, width), jnp.float32)]
    if with_deg:
        out_type.append(jax.ShapeDtypeStruct((NC * NP,), jnp.float32))
    scratch = [
        pltpu.VMEM((K, CHUNK), jnp.int32),
        pltpu.VMEM((K, CHUNK), jnp.int32),
        pltpu.VMEM((K, CHUNK, width), jnp.float32),
    ]
    if with_deg:
        scratch += [
            pltpu.VMEM((CHUNK,), jnp.float32),
        ]
    scratch += [pltpu.VMEM_SHARED((NP, width), jnp.float32)]
    if with_deg:
        scratch += [pltpu.VMEM_SHARED((NP,), jnp.float32)]
    scratch += [pltpu.SemaphoreType.DMA, pltpu.SemaphoreType.DMA]

    return pl.kernel(
        functools.partial(_seg_body, width, with_deg, K),
        out_type=tuple(out_type),
        mesh=mesh,
        scratch_types=scratch,
        compiler_params=pltpu.CompilerParams(use_tc_tiling_on_sc=False),
    )


def _dense1_body(pref, iref, xref, w1l, w1r, b1r, w2l, w2r, b2r, zref, hrref):
    mean = (pref[0] + pref[1]) * iref[...]
    h = jnp.dot(mean, w1l[...], preferred_element_type=jnp.float32)
    h += jnp.dot(xref[...], w1r[...], preferred_element_type=jnp.float32)
    h = jnp.maximum(h + b1r[...], 0.0)
    zref[...] = jnp.dot(h, w2l[...], preferred_element_type=jnp.float32)
    hrref[...] = jnp.dot(h, w2r[...], preferred_element_type=jnp.float32) + b2r[...]


def _dense2_body(aref, iref, hrref, oref):
    mean = (aref[0] + aref[1]) * iref[...]
    oref[...] = mean + hrref[...]


_full = lambda *dims: pl.BlockSpec(dims, lambda i: (0,) * len(dims))


def kernel(feature, edge_index, edge_type, W1_l, W1_r, b1, W2_l, W2_r, b2):
    del edge_type
    edges2d = edge_index.reshape(2, NCH, CHUNK)

    zagg16 = jnp.zeros((PER_TILE, D_IN), jnp.float32)
    ones_e = jnp.ones((CHUNK,), jnp.float32)
    zeros_d = jnp.zeros((PER_TILE,), jnp.float32)
    agg1_p, deg_flat = _make_seg_kernel(D_IN, True, K1)(
        feature, edges2d, zagg16, ones_e, zeros_d)
    a1r = agg1_p.reshape(NC, NP * D_IN // 128, 128)
    deg = deg_flat.reshape(NC, NP).sum(axis=0)
    invr = jnp.repeat(1.0 / jnp.maximum(deg, 1.0), D_IN
                      ).reshape(NP * D_IN // 128, 128)
    xr = feature.reshape(N * D_IN // 128, 128)

    w2l_p = jnp.zeros((D_HID, Z_W), jnp.float32).at[:, :3].set(W2_l)
    w2r_p = jnp.zeros((D_HID, HR_W), jnp.float32).at[:, :3].set(W2_r)
    b1r = b1.reshape(1, D_HID)
    b2r = jnp.zeros((1, HR_W), jnp.float32).at[0, :3].set(b2)

    PKB = ROWBLK * D_IN // 128
    eye8 = jnp.eye(8, dtype=jnp.float32)
    w1l_b = jnp.kron(eye8, W1_l)
    w1r_b = jnp.kron(eye8, W1_r)
    b1_b = jnp.tile(b1, 8).reshape(1, 8 * D_HID)
    w2l_b = jnp.kron(eye8, w2l_p)
    w2r_b = jnp.kron(eye8, w2r_p)
    b2_b = jnp.tile(b2r[0], 8).reshape(1, 8 * HR_W)
    z_pk, hr_pk = pl.pallas_call(
        _dense1_body,
        grid=(NBLK,),
        in_specs=[
            pl.BlockSpec((NC, PKB, 128), lambda i: (0, i, 0)),
            pl.BlockSpec((PKB, 128), lambda i: (i, 0)),
            pl.BlockSpec((PKB, 128), lambda i: (i, 0)),
            _full(128, 8 * D_HID), _full(128, 8 * D_HID), _full(1, 8 * D_HID),
            _full(8 * D_HID, 8 * Z_W), _full(8 * D_HID, 8 * HR_W),
            _full(1, 8 * HR_W),
        ],
        out_specs=[
            pl.BlockSpec((PKB, 8 * Z_W), lambda i: (i, 0)),
            pl.BlockSpec((PKB, 8 * HR_W), lambda i: (i, 0)),
        ],
        out_shape=[
            jax.ShapeDtypeStruct((NP * D_IN // 128, 8 * Z_W), jnp.float32),
            jax.ShapeDtypeStruct((NP * D_IN // 128, 8 * HR_W), jnp.float32),
        ],
    )(a1r, invr, xr, w1l_b, w1r_b, b1_b, w2l_b, w2r_b, b2_b)
    z = z_pk.reshape(NP, Z_W)

    zagg_z = jnp.zeros((PER_TILE, Z_W), jnp.float32)
    (agg2_p,) = _make_seg_kernel(Z_W, False, K2)(z, edges2d, zagg_z)

    a2r = agg2_p.reshape(NC, NP * D_IN // 128, 8 * Z_W)
    invr8 = jnp.repeat(1.0 / jnp.maximum(deg, 1.0), Z_W
                       ).reshape(NP * D_IN // 128, 8 * Z_W)
    out_pk = pl.pallas_call(
        _dense2_body,
        grid=(NBLK,),
        in_specs=[
            pl.BlockSpec((NC, PKB, 8 * Z_W), lambda i: (0, i, 0)),
            pl.BlockSpec((PKB, 8 * Z_W), lambda i: (i, 0)),
            pl.BlockSpec((PKB, 8 * HR_W), lambda i: (i, 0)),
        ],
        out_specs=pl.BlockSpec((PKB, 8 * HR_W), lambda i: (i, 0)),
        out_shape=jax.ShapeDtypeStruct((NP * D_IN // 128, 8 * HR_W),
                                       jnp.float32),
    )(a2r, invr8, hr_pk)

    return out_pk.reshape(NP, HR_W)[:N, :3]
